# baseline (speedup 1.0000x reference)
import functools

import jax
import jax.numpy as jnp
from jax import lax
from jax.experimental import pallas as pl
from jax.experimental.pallas import tpu as pltpu
from jax.experimental.pallas import tpu_sc as plsc

N = 100000
D = 128
B = 512
EPS = 1e-05
R = 2000

NC = 2
NS = 16
NW = NC * NS
KPT = 3136
NP = NW * KPT
CW = 98
RT = KPT // CW
NROWS = NP // CW
BB = 520
GIT = KPT // 16

_MESH = plsc.VectorSubcoreMesh(core_axis_name="c", subcore_axis_name="s")


def _rowstats_body(x_ref, rs_ref, rq_ref):
    x = x_ref[...]
    rs_ref[...] = jnp.sum(x, axis=1, keepdims=True)
    rq_ref[...] = jnp.sum(x * x, axis=1, keepdims=True)


@functools.partial(
    pl.kernel,
    mesh=_MESH,
    out_type=[jax.ShapeDtypeStruct((NC, BB), jnp.float32)] * 3,
    scratch_types=[
        pltpu.VMEM((RT, CW), jnp.int32),
        pltpu.VMEM((RT, CW), jnp.float32),
        pltpu.VMEM((RT, CW), jnp.float32),
        pltpu.VMEM((RT, CW), jnp.float32),
        pltpu.VMEM_SHARED((BB,), jnp.float32),
        pltpu.VMEM_SHARED((BB,), jnp.float32),
        pltpu.VMEM_SHARED((BB,), jnp.float32),
    ],
)
def _sc_scatter(bidx_hbm, ones_hbm, rs_hbm, rq_hbm, zeros_hbm,
                pc_hbm, ps_hbm, pq_hbm,
                idx_v, ones_v, rs_v, rq_v, sh_c, sh_s, sh_q):
    cid = lax.axis_index("c")
    sid = lax.axis_index("s")
    wid = sid * NC + cid
    pltpu.sync_copy(bidx_hbm.at[pl.ds(wid * RT, RT)], idx_v)
    pltpu.sync_copy(ones_hbm, ones_v)
    pltpu.sync_copy(rs_hbm.at[pl.ds(wid * RT, RT)], rs_v)
    pltpu.sync_copy(rq_hbm.at[pl.ds(wid * RT, RT)], rq_v)

    @pl.when(sid == 0)
    def _zero():
        pltpu.sync_copy(zeros_hbm, sh_c)
        pltpu.sync_copy(zeros_hbm, sh_s)
        pltpu.sync_copy(zeros_hbm, sh_q)

    plsc.subcore_barrier()
    for j in range(RT):
        pltpu.sync_copy(ones_v.at[j], sh_c.at[idx_v.at[j]], add=True)
        pltpu.sync_copy(rs_v.at[j], sh_s.at[idx_v.at[j]], add=True)
        pltpu.sync_copy(rq_v.at[j], sh_q.at[idx_v.at[j]], add=True)
    plsc.subcore_barrier()

    @pl.when(sid == 0)
    def _writeout():
        pltpu.sync_copy(sh_c, pc_hbm.at[cid])
        pltpu.sync_copy(sh_s, ps_hbm.at[cid])
        pltpu.sync_copy(sh_q, pq_hbm.at[cid])


@functools.partial(
    pl.kernel,
    mesh=_MESH,
    out_type=[jax.ShapeDtypeStruct((NP,), jnp.float32)] * 3,
    scratch_types=[
        pltpu.VMEM((KPT,), jnp.int32),
        pltpu.VMEM((KPT,), jnp.float32),
        pltpu.VMEM((KPT,), jnp.float32),
        pltpu.VMEM((KPT,), jnp.float32),
        pltpu.SemaphoreType.DMA,
    ],
)
def _sc_gather(bflat_hbm, bc_hbm, bs_hbm, bq_hbm, outc, outs, outq,
               bidx_v, c_v, s_v, q_v, sem):
    cid = lax.axis_index("c")
    sid = lax.axis_index("s")
    wid = sid * NC + cid
    base = wid * KPT
    pltpu.sync_copy(bflat_hbm.at[pl.ds(base, KPT)], bidx_v)
    pltpu.async_copy(bc_hbm.at[bidx_v], c_v, sem).wait()
    pltpu.async_copy(bs_hbm.at[bidx_v], s_v, sem).wait()
    pltpu.async_copy(bq_hbm.at[bidx_v], q_v, sem).wait()
    pltpu.sync_copy(c_v, outc.at[pl.ds(base, KPT)])
    pltpu.sync_copy(s_v, outs.at[pl.ds(base, KPT)])
    pltpu.sync_copy(q_v, outq.at[pl.ds(base, KPT)])


def _norm_body(x_ref, c_ref, s_ref, q_ref, w_ref, b_ref, out_ref):
    x = x_ref[...]
    c = c_ref[...]
    s = s_ref[...]
    q = q_ref[...]
    norm = jnp.maximum(c, 1.0) * float(D)
    mean = s / norm
    var = jnp.maximum(q / norm - mean * mean, 0.0)
    inv = 1.0 / (jnp.sqrt(var) + EPS)
    out_ref[...] = (x - mean) * inv * w_ref[...] + b_ref[...]


@jax.jit
def kernel(x, weight, bias, batch):
    b32 = batch.astype(jnp.int32)
    bp = jnp.pad(b32, (0, NP - N), constant_values=B)
    bp2 = bp.reshape(NROWS, CW)
    w2 = weight.reshape(1, D)
    bias2 = bias.reshape(1, D)
    grid = N // R

    rs, rq = pl.pallas_call(
        _rowstats_body,
        grid=(grid,),
        in_specs=[pl.BlockSpec((R, D), lambda i: (i, 0))],
        out_specs=[
            pl.BlockSpec((R, 1), lambda i: (i, 0)),
            pl.BlockSpec((R, 1), lambda i: (i, 0)),
        ],
        out_shape=[
            jax.ShapeDtypeStruct((NP, 1), jnp.float32),
            jax.ShapeDtypeStruct((NP, 1), jnp.float32),
        ],
    )(x)

    ones_t = jnp.ones((RT, CW), jnp.float32)
    zeros_t = jnp.zeros((BB,), jnp.float32)
    pc, ps, pq = _sc_scatter(bp2, ones_t, rs.reshape(NROWS, CW),
                             rq.reshape(NROWS, CW), zeros_t)
    bins_c = pc[0] + pc[1]
    bins_s = ps[0] + ps[1]
    bins_q = pq[0] + pq[1]
    crow, srow, qrow = _sc_gather(bp, bins_c, bins_s, bins_q)

    out = pl.pallas_call(
        _norm_body,
        grid=(grid,),
        in_specs=[
            pl.BlockSpec((R, D), lambda i: (i, 0)),
            pl.BlockSpec((R, 1), lambda i: (i, 0)),
            pl.BlockSpec((R, 1), lambda i: (i, 0)),
            pl.BlockSpec((R, 1), lambda i: (i, 0)),
            pl.BlockSpec((1, D), lambda i: (0, 0)),
            pl.BlockSpec((1, D), lambda i: (0, 0)),
        ],
        out_specs=pl.BlockSpec((R, D), lambda i: (i, 0)),
        out_shape=jax.ShapeDtypeStruct((N, D), jnp.float32),
    )(x, crow.reshape(NP, 1), srow.reshape(NP, 1), qrow.reshape(NP, 1),
      w2, bias2)
    return out


# speedup vs baseline: 1.0030x; 1.0030x over previous
import functools

import jax
import jax.numpy as jnp
from jax import lax
from jax.experimental import pallas as pl
from jax.experimental.pallas import tpu as pltpu
from jax.experimental.pallas import tpu_sc as plsc

N = 100000
D = 128
B = 512
EPS = 1e-05
R = 2000

NC = 2
NS = 16
NW = NC * NS
KPT = 3136
NP = NW * KPT
CW = 98
RT = KPT // CW
NROWS = NP // CW
BB = 520
GIT = KPT // 16

_MESH = plsc.VectorSubcoreMesh(core_axis_name="c", subcore_axis_name="s")


def _rowstats_body(x_ref, rs_ref, rq_ref):
    x = x_ref[...]
    rs_ref[...] = jnp.sum(x, axis=1, keepdims=True)
    rq_ref[...] = jnp.sum(x * x, axis=1, keepdims=True)


@functools.partial(
    pl.kernel,
    mesh=_MESH,
    out_type=[jax.ShapeDtypeStruct((NC, BB), jnp.float32)] * 3,
    scratch_types=[
        pltpu.VMEM((RT, CW), jnp.int32),
        pltpu.VMEM((RT, CW), jnp.float32),
        pltpu.VMEM((RT, CW), jnp.float32),
        pltpu.VMEM((RT, CW), jnp.float32),
        pltpu.VMEM_SHARED((BB,), jnp.float32),
        pltpu.VMEM_SHARED((BB,), jnp.float32),
        pltpu.VMEM_SHARED((BB,), jnp.float32),
        pltpu.SemaphoreType.DMA,
    ],
)
def _sc_scatter(bidx_hbm, ones_hbm, rs_hbm, rq_hbm, zeros_hbm,
                pc_hbm, ps_hbm, pq_hbm,
                idx_v, ones_v, rs_v, rq_v, sh_c, sh_s, sh_q, sem):
    cid = lax.axis_index("c")
    sid = lax.axis_index("s")
    wid = sid * NC + cid
    stg = [
        pltpu.async_copy(bidx_hbm.at[pl.ds(wid * RT, RT)], idx_v, sem),
        pltpu.async_copy(ones_hbm, ones_v, sem),
        pltpu.async_copy(rs_hbm.at[pl.ds(wid * RT, RT)], rs_v, sem),
        pltpu.async_copy(rq_hbm.at[pl.ds(wid * RT, RT)], rq_v, sem),
    ]
    for cdesc in stg:
        cdesc.wait()

    @pl.when(sid == 0)
    def _zero():
        pltpu.sync_copy(zeros_hbm, sh_c)
        pltpu.sync_copy(zeros_hbm, sh_s)
        pltpu.sync_copy(zeros_hbm, sh_q)

    plsc.subcore_barrier()
    descs = []
    for j in range(RT):
        descs.append(
            pltpu.async_copy(ones_v.at[j], sh_c.at[idx_v.at[j]], sem,
                             add=True))
        descs.append(
            pltpu.async_copy(rs_v.at[j], sh_s.at[idx_v.at[j]], sem,
                             add=True))
        descs.append(
            pltpu.async_copy(rq_v.at[j], sh_q.at[idx_v.at[j]], sem,
                             add=True))
    for cdesc in descs:
        cdesc.wait()
    plsc.subcore_barrier()

    @pl.when(sid == 0)
    def _writeout():
        pltpu.sync_copy(sh_c, pc_hbm.at[cid])
        pltpu.sync_copy(sh_s, ps_hbm.at[cid])
        pltpu.sync_copy(sh_q, pq_hbm.at[cid])


@functools.partial(
    pl.kernel,
    mesh=_MESH,
    out_type=[jax.ShapeDtypeStruct((NP,), jnp.float32)] * 3,
    scratch_types=[
        pltpu.VMEM((KPT,), jnp.int32),
        pltpu.VMEM((KPT,), jnp.float32),
        pltpu.VMEM((KPT,), jnp.float32),
        pltpu.VMEM((KPT,), jnp.float32),
        pltpu.SemaphoreType.DMA,
    ],
)
def _sc_gather(bflat_hbm, bc_hbm, bs_hbm, bq_hbm, outc, outs, outq,
               bidx_v, c_v, s_v, q_v, sem):
    cid = lax.axis_index("c")
    sid = lax.axis_index("s")
    wid = sid * NC + cid
    base = wid * KPT
    pltpu.sync_copy(bflat_hbm.at[pl.ds(base, KPT)], bidx_v)
    pltpu.async_copy(bc_hbm.at[bidx_v], c_v, sem).wait()
    pltpu.async_copy(bs_hbm.at[bidx_v], s_v, sem).wait()
    pltpu.async_copy(bq_hbm.at[bidx_v], q_v, sem).wait()
    pltpu.sync_copy(c_v, outc.at[pl.ds(base, KPT)])
    pltpu.sync_copy(s_v, outs.at[pl.ds(base, KPT)])
    pltpu.sync_copy(q_v, outq.at[pl.ds(base, KPT)])


def _norm_body(x_ref, c_ref, s_ref, q_ref, w_ref, b_ref, out_ref):
    x = x_ref[...]
    c = c_ref[...]
    s = s_ref[...]
    q = q_ref[...]
    norm = jnp.maximum(c, 1.0) * float(D)
    mean = s / norm
    var = jnp.maximum(q / norm - mean * mean, 0.0)
    inv = 1.0 / (jnp.sqrt(var) + EPS)
    out_ref[...] = (x - mean) * inv * w_ref[...] + b_ref[...]


@jax.jit
def kernel(x, weight, bias, batch):
    b32 = batch.astype(jnp.int32)
    bp = jnp.pad(b32, (0, NP - N), constant_values=B)
    bp2 = bp.reshape(NROWS, CW)
    w2 = weight.reshape(1, D)
    bias2 = bias.reshape(1, D)
    grid = N // R

    rs, rq = pl.pallas_call(
        _rowstats_body,
        grid=(grid,),
        in_specs=[pl.BlockSpec((R, D), lambda i: (i, 0))],
        out_specs=[
            pl.BlockSpec((R, 1), lambda i: (i, 0)),
            pl.BlockSpec((R, 1), lambda i: (i, 0)),
        ],
        out_shape=[
            jax.ShapeDtypeStruct((NP, 1), jnp.float32),
            jax.ShapeDtypeStruct((NP, 1), jnp.float32),
        ],
    )(x)

    ones_t = jnp.ones((RT, CW), jnp.float32)
    zeros_t = jnp.zeros((BB,), jnp.float32)
    pc, ps, pq = _sc_scatter(bp2, ones_t, rs.reshape(NROWS, CW),
                             rq.reshape(NROWS, CW), zeros_t)
    bins_c = pc[0] + pc[1]
    bins_s = ps[0] + ps[1]
    bins_q = pq[0] + pq[1]
    crow, srow, qrow = _sc_gather(bp, bins_c, bins_s, bins_q)

    out = pl.pallas_call(
        _norm_body,
        grid=(grid,),
        in_specs=[
            pl.BlockSpec((R, D), lambda i: (i, 0)),
            pl.BlockSpec((R, 1), lambda i: (i, 0)),
            pl.BlockSpec((R, 1), lambda i: (i, 0)),
            pl.BlockSpec((R, 1), lambda i: (i, 0)),
            pl.BlockSpec((1, D), lambda i: (0, 0)),
            pl.BlockSpec((1, D), lambda i: (0, 0)),
        ],
        out_specs=pl.BlockSpec((R, D), lambda i: (i, 0)),
        out_shape=jax.ShapeDtypeStruct((N, D), jnp.float32),
    )(x, crow.reshape(NP, 1), srow.reshape(NP, 1), qrow.reshape(NP, 1),
      w2, bias2)
    return out


# speedup vs baseline: 1.0123x; 1.0093x over previous
import functools

import jax
import jax.numpy as jnp
from jax import lax
from jax.experimental import pallas as pl
from jax.experimental.pallas import tpu as pltpu
from jax.experimental.pallas import tpu_sc as plsc

N = 100000
D = 128
B = 512
EPS = 1e-05
R = 2000

NC = 2
NS = 16
NW = NC * NS
KPT = 3136
NP = NW * KPT
CW = 98
RT = KPT // CW
NROWS = NP // CW
BB = 520
GIT = KPT // 16

_MESH = plsc.VectorSubcoreMesh(core_axis_name="c", subcore_axis_name="s")


def _rowstats_body(x_ref, rs_ref, rq_ref):
    x = x_ref[...]
    rs_ref[...] = jnp.sum(x, axis=1, keepdims=True)
    rq_ref[...] = jnp.sum(x * x, axis=1, keepdims=True)


@functools.partial(
    pl.kernel,
    mesh=_MESH,
    out_type=[jax.ShapeDtypeStruct((NC, BB), jnp.float32)] * 3,
    scratch_types=[
        pltpu.VMEM((KPT,), jnp.int32),
        pltpu.VMEM((KPT,), jnp.float32),
        pltpu.VMEM((KPT,), jnp.float32),
        pltpu.VMEM((KPT,), jnp.float32),
        pltpu.VMEM_SHARED((BB,), jnp.float32),
        pltpu.VMEM_SHARED((BB,), jnp.float32),
        pltpu.VMEM_SHARED((BB,), jnp.float32),
        pltpu.SemaphoreType.DMA,
    ],
)
def _sc_scatter(bidx_hbm, ones_hbm, rs_hbm, rq_hbm, zeros_hbm,
                pc_hbm, ps_hbm, pq_hbm,
                idx_v, ones_v, rs_v, rq_v, sh_c, sh_s, sh_q, sem):
    cid = lax.axis_index("c")
    sid = lax.axis_index("s")
    wid = sid * NC + cid
    base = wid * KPT
    stg = [
        pltpu.async_copy(bidx_hbm.at[pl.ds(base, KPT)], idx_v, sem),
        pltpu.async_copy(ones_hbm, ones_v, sem),
        pltpu.async_copy(rs_hbm.at[pl.ds(base, KPT)], rs_v, sem),
        pltpu.async_copy(rq_hbm.at[pl.ds(base, KPT)], rq_v, sem),
    ]
    for cdesc in stg:
        cdesc.wait()

    @pl.when(sid == 0)
    def _zero():
        pltpu.sync_copy(zeros_hbm, sh_c)
        pltpu.sync_copy(zeros_hbm, sh_s)
        pltpu.sync_copy(zeros_hbm, sh_q)

    plsc.subcore_barrier()
    descs = [
        pltpu.async_copy(ones_v, sh_c.at[idx_v], sem, add=True),
        pltpu.async_copy(rs_v, sh_s.at[idx_v], sem, add=True),
        pltpu.async_copy(rq_v, sh_q.at[idx_v], sem, add=True),
    ]
    for cdesc in descs:
        cdesc.wait()
    plsc.subcore_barrier()

    @pl.when(sid == 0)
    def _writeout():
        pltpu.sync_copy(sh_c, pc_hbm.at[cid])
        pltpu.sync_copy(sh_s, ps_hbm.at[cid])
        pltpu.sync_copy(sh_q, pq_hbm.at[cid])


@functools.partial(
    pl.kernel,
    mesh=_MESH,
    out_type=[jax.ShapeDtypeStruct((NP,), jnp.float32)] * 3,
    scratch_types=[
        pltpu.VMEM((KPT,), jnp.int32),
        pltpu.VMEM((KPT,), jnp.float32),
        pltpu.VMEM((KPT,), jnp.float32),
        pltpu.VMEM((KPT,), jnp.float32),
        pltpu.SemaphoreType.DMA,
    ],
)
def _sc_gather(bflat_hbm, bc_hbm, bs_hbm, bq_hbm, outc, outs, outq,
               bidx_v, c_v, s_v, q_v, sem):
    cid = lax.axis_index("c")
    sid = lax.axis_index("s")
    wid = sid * NC + cid
    base = wid * KPT
    pltpu.sync_copy(bflat_hbm.at[pl.ds(base, KPT)], bidx_v)
    pltpu.async_copy(bc_hbm.at[bidx_v], c_v, sem).wait()
    pltpu.async_copy(bs_hbm.at[bidx_v], s_v, sem).wait()
    pltpu.async_copy(bq_hbm.at[bidx_v], q_v, sem).wait()
    pltpu.sync_copy(c_v, outc.at[pl.ds(base, KPT)])
    pltpu.sync_copy(s_v, outs.at[pl.ds(base, KPT)])
    pltpu.sync_copy(q_v, outq.at[pl.ds(base, KPT)])


def _norm_body(x_ref, c_ref, s_ref, q_ref, w_ref, b_ref, out_ref):
    x = x_ref[...]
    c = c_ref[...]
    s = s_ref[...]
    q = q_ref[...]
    norm = jnp.maximum(c, 1.0) * float(D)
    mean = s / norm
    var = jnp.maximum(q / norm - mean * mean, 0.0)
    inv = 1.0 / (jnp.sqrt(var) + EPS)
    out_ref[...] = (x - mean) * inv * w_ref[...] + b_ref[...]


@jax.jit
def kernel(x, weight, bias, batch):
    b32 = batch.astype(jnp.int32)
    bp = jnp.pad(b32, (0, NP - N), constant_values=B)
    bp2 = bp.reshape(NROWS, CW)
    w2 = weight.reshape(1, D)
    bias2 = bias.reshape(1, D)
    grid = N // R

    rs, rq = pl.pallas_call(
        _rowstats_body,
        grid=(grid,),
        in_specs=[pl.BlockSpec((R, D), lambda i: (i, 0))],
        out_specs=[
            pl.BlockSpec((R, 1), lambda i: (i, 0)),
            pl.BlockSpec((R, 1), lambda i: (i, 0)),
        ],
        out_shape=[
            jax.ShapeDtypeStruct((NP, 1), jnp.float32),
            jax.ShapeDtypeStruct((NP, 1), jnp.float32),
        ],
    )(x)

    ones_t = jnp.ones((KPT,), jnp.float32)
    zeros_t = jnp.zeros((BB,), jnp.float32)
    pc, ps, pq = _sc_scatter(bp, ones_t, rs.reshape(NP), rq.reshape(NP),
                             zeros_t)
    bins_c = pc[0] + pc[1]
    bins_s = ps[0] + ps[1]
    bins_q = pq[0] + pq[1]
    crow, srow, qrow = _sc_gather(bp, bins_c, bins_s, bins_q)

    out = pl.pallas_call(
        _norm_body,
        grid=(grid,),
        in_specs=[
            pl.BlockSpec((R, D), lambda i: (i, 0)),
            pl.BlockSpec((R, 1), lambda i: (i, 0)),
            pl.BlockSpec((R, 1), lambda i: (i, 0)),
            pl.BlockSpec((R, 1), lambda i: (i, 0)),
            pl.BlockSpec((1, D), lambda i: (0, 0)),
            pl.BlockSpec((1, D), lambda i: (0, 0)),
        ],
        out_specs=pl.BlockSpec((R, D), lambda i: (i, 0)),
        out_shape=jax.ShapeDtypeStruct((N, D), jnp.float32),
    )(x, crow.reshape(NP, 1), srow.reshape(NP, 1), qrow.reshape(NP, 1),
      w2, bias2)
    return out


# speedup vs baseline: 1.6797x; 1.6593x over previous
import functools

import jax
import jax.numpy as jnp
from jax import lax
from jax.experimental import pallas as pl
from jax.experimental.pallas import tpu as pltpu
from jax.experimental.pallas import tpu_sc as plsc

N = 100000
D = 128
B = 512
EPS = 1e-05
R = 2000

NC = 2
NS = 16
NW = NC * NS
KPT = 3136
NP = NW * KPT
BB = 520
R2C = 2048
G2 = NP // R2C

_MESH = plsc.VectorSubcoreMesh(core_axis_name="c", subcore_axis_name="s")


def _pass1_body(x_ref, brow_ref, acc_ref, mean_ref, inv_ref):
    i = pl.program_id(0)
    x = x_ref[...]
    rs = jnp.sum(x, axis=1, keepdims=True)
    rq = jnp.sum(x * x, axis=1, keepdims=True)
    ones = jnp.ones((R, 1), jnp.float32)
    vals = jnp.concatenate([ones, rs, rq, jnp.zeros((R, 1), jnp.float32)],
                           axis=1)
    b_row = brow_ref[0]
    seg_ids = jax.lax.broadcasted_iota(jnp.int32, (B, R), 0)
    ohT = (seg_ids == b_row).astype(jnp.float32)

    @pl.when(i == 0)
    def _init():
        acc_ref[...] = jnp.zeros_like(acc_ref)

    acc_ref[...] += jax.lax.dot(ohT, vals, preferred_element_type=jnp.float32)

    @pl.when(i == pl.num_programs(0) - 1)
    def _finalize():
        acc = acc_ref[...]
        cnt = acc[:, 0:1]
        s = acc[:, 1:2]
        q = acc[:, 2:3]
        norm = jnp.maximum(cnt, 1.0) * float(D)
        mean = s / norm
        var = jnp.maximum(q / norm - mean * mean, 0.0)
        inv = 1.0 / (jnp.sqrt(var) + EPS)
        mean_ref[0:B, :] = mean
        mean_ref[B:BB, :] = jnp.zeros((BB - B, 1), jnp.float32)
        inv_ref[0:B, :] = inv
        inv_ref[B:BB, :] = jnp.zeros((BB - B, 1), jnp.float32)


@functools.partial(
    pl.kernel,
    mesh=_MESH,
    out_type=[jax.ShapeDtypeStruct((NP,), jnp.float32)] * 2,
    scratch_types=[
        pltpu.VMEM((KPT,), jnp.int32),
        pltpu.VMEM((KPT,), jnp.float32),
        pltpu.VMEM((KPT,), jnp.float32),
        pltpu.SemaphoreType.DMA,
    ],
)
def _sc_gather(bflat_hbm, mt_hbm, it_hbm, outm, outi, bidx_v, m_v, i_v, sem):
    cid = lax.axis_index("c")
    sid = lax.axis_index("s")
    wid = sid * NC + cid
    base = wid * KPT
    pltpu.sync_copy(bflat_hbm.at[pl.ds(base, KPT)], bidx_v)
    pltpu.async_copy(mt_hbm.at[bidx_v], m_v, sem).wait()
    pltpu.async_copy(it_hbm.at[bidx_v], i_v, sem).wait()
    pltpu.sync_copy(m_v, outm.at[pl.ds(base, KPT)])
    pltpu.sync_copy(i_v, outi.at[pl.ds(base, KPT)])


def _pass2_body(x_ref, m_ref, i_ref, w_ref, b_ref, out_ref):
    x = x_ref[...]
    m_col = m_ref[0].reshape(R2C, 1)
    i_col = i_ref[0].reshape(R2C, 1)
    out_ref[...] = (x - m_col) * i_col * w_ref[...] + b_ref[...]


@jax.jit
def kernel(x, weight, bias, batch):
    b32 = batch.astype(jnp.int32)
    bp = jnp.pad(b32, (0, NP - N), constant_values=B)
    brow = b32.reshape(N // R, 1, R)
    w2 = weight.reshape(1, D)
    bias2 = bias.reshape(1, D)

    _, meant, invt = pl.pallas_call(
        _pass1_body,
        grid=(N // R,),
        in_specs=[
            pl.BlockSpec((R, D), lambda i: (i, 0)),
            pl.BlockSpec((1, 1, R), lambda i: (i, 0, 0)),
        ],
        out_specs=[
            pl.BlockSpec((B, 4), lambda i: (0, 0)),
            pl.BlockSpec((BB, 1), lambda i: (0, 0)),
            pl.BlockSpec((BB, 1), lambda i: (0, 0)),
        ],
        out_shape=[
            jax.ShapeDtypeStruct((B, 4), jnp.float32),
            jax.ShapeDtypeStruct((BB, 1), jnp.float32),
            jax.ShapeDtypeStruct((BB, 1), jnp.float32),
        ],
    )(x, brow)

    mrow, irow = _sc_gather(bp, meant.reshape(BB), invt.reshape(BB))

    out = pl.pallas_call(
        _pass2_body,
        grid=(G2,),
        in_specs=[
            pl.BlockSpec((R2C, D), lambda i: (i, 0)),
            pl.BlockSpec((1, 1, R2C), lambda i: (i, 0, 0)),
            pl.BlockSpec((1, 1, R2C), lambda i: (i, 0, 0)),
            pl.BlockSpec((1, D), lambda i: (0, 0)),
            pl.BlockSpec((1, D), lambda i: (0, 0)),
        ],
        out_specs=pl.BlockSpec((R2C, D), lambda i: (i, 0)),
        out_shape=jax.ShapeDtypeStruct((N, D), jnp.float32),
    )(x, mrow.reshape(G2, 1, R2C), irow.reshape(G2, 1, R2C), w2, bias2)
    return out


# speedup vs baseline: 3.3604x; 2.0006x over previous
import jax
import jax.numpy as jnp
from jax.experimental import pallas as pl

N = 100000
D = 128
B = 512
EPS = 1e-05
R = 2000


def _pass1_body(x_ref, brow_ref, acc_ref, mi_ref):
    i = pl.program_id(0)
    x = x_ref[...]
    rs = jnp.sum(x, axis=1, keepdims=True)
    rq = jnp.sum(x * x, axis=1, keepdims=True)
    ones = jnp.ones((R, 1), jnp.float32)
    vals = jnp.concatenate([ones, rs, rq, jnp.zeros((R, 1), jnp.float32)],
                           axis=1)
    b_row = brow_ref[0]
    seg_ids = jax.lax.broadcasted_iota(jnp.int32, (B, R), 0)
    ohT = (seg_ids == b_row).astype(jnp.bfloat16)

    @pl.when(i == 0)
    def _init():
        acc_ref[...] = jnp.zeros_like(acc_ref)

    acc_ref[...] += jax.lax.dot(ohT, vals.astype(jnp.bfloat16),
                                preferred_element_type=jnp.float32)

    @pl.when(i == pl.num_programs(0) - 1)
    def _finalize():
        acc = acc_ref[...]
        cnt = acc[:, 0:1]
        s = acc[:, 1:2]
        q = acc[:, 2:3]
        norm = jnp.maximum(cnt, 1.0) * float(D)
        mean = s / norm
        var = jnp.maximum(q / norm - mean * mean, 0.0)
        inv = 1.0 / (jnp.sqrt(var) + EPS)
        mi_ref[...] = jnp.concatenate([mean, inv], axis=1)


def _pass2_body(x_ref, bcol_ref, w_ref, bias_ref, mi_ref, out_ref):
    x = x_ref[...]
    b_col = bcol_ref[...]
    seg_ids = jax.lax.broadcasted_iota(jnp.int32, (R, B), 1)
    oh = (seg_ids == b_col).astype(jnp.float32)
    g = jax.lax.dot(oh, mi_ref[...], preferred_element_type=jnp.float32)
    mean = g[:, 0:1]
    inv = g[:, 1:2]
    out_ref[...] = (x - mean) * inv * w_ref[...] + bias_ref[...]


@jax.jit
def kernel(x, weight, bias, batch):
    b32 = batch.astype(jnp.int32)
    brow = b32.reshape(N // R, 1, R)
    bcol = b32.reshape(N, 1)
    w2 = weight.reshape(1, D)
    bias2 = bias.reshape(1, D)
    grid = N // R

    acc, mi = pl.pallas_call(
        _pass1_body,
        grid=(grid,),
        in_specs=[
            pl.BlockSpec((R, D), lambda i: (i, 0)),
            pl.BlockSpec((1, 1, R), lambda i: (i, 0, 0)),
        ],
        out_specs=[
            pl.BlockSpec((B, 4), lambda i: (0, 0)),
            pl.BlockSpec((B, 2), lambda i: (0, 0)),
        ],
        out_shape=[
            jax.ShapeDtypeStruct((B, 4), jnp.float32),
            jax.ShapeDtypeStruct((B, 2), jnp.float32),
        ],
    )(x, brow)

    out = pl.pallas_call(
        _pass2_body,
        grid=(grid,),
        in_specs=[
            pl.BlockSpec((R, D), lambda i: (i, 0)),
            pl.BlockSpec((R, 1), lambda i: (i, 0)),
            pl.BlockSpec((1, D), lambda i: (0, 0)),
            pl.BlockSpec((1, D), lambda i: (0, 0)),
            pl.BlockSpec((B, 2), lambda i: (0, 0)),
        ],
        out_specs=pl.BlockSpec((R, D), lambda i: (i, 0)),
        out_shape=jax.ShapeDtypeStruct((N, D), jnp.float32),
    )(x, bcol, w2, bias2, mi)
    return out


# speedup vs baseline: 3.8050x; 1.1323x over previous
import jax
import jax.numpy as jnp
from jax.experimental import pallas as pl

N = 100000
D = 128
B = 512
EPS = 1e-05
R = 4000


def _pass1_body(x_ref, brow_ref, acc_ref, mi_ref):
    i = pl.program_id(0)
    x = x_ref[...]
    rs = jnp.sum(x, axis=1, keepdims=True)
    rq = jnp.sum(x * x, axis=1, keepdims=True)
    ones = jnp.ones((R, 1), jnp.float32)
    vals = jnp.concatenate([ones, rs, rq, jnp.zeros((R, 1), jnp.float32)],
                           axis=1)
    b_row = brow_ref[0]
    seg_ids = jax.lax.broadcasted_iota(jnp.int32, (B, R), 0)
    ohT = (seg_ids == b_row).astype(jnp.float32)

    @pl.when(i == 0)
    def _init():
        acc_ref[...] = jnp.zeros_like(acc_ref)

    acc_ref[...] += jax.lax.dot(ohT, vals, preferred_element_type=jnp.float32)

    @pl.when(i == pl.num_programs(0) - 1)
    def _finalize():
        acc = acc_ref[...]
        cnt = acc[:, 0:1]
        s = acc[:, 1:2]
        q = acc[:, 2:3]
        norm = jnp.maximum(cnt, 1.0) * float(D)
        mean = s / norm
        var = jnp.maximum(q / norm - mean * mean, 0.0)
        inv = 1.0 / (jnp.sqrt(var) + EPS)
        mi_ref[...] = jnp.concatenate([mean, inv], axis=1)


def _pass2_body(x_ref, bcol_ref, w_ref, bias_ref, mi_ref, out_ref):
    x = x_ref[...]
    b_col = bcol_ref[...]
    seg_ids = jax.lax.broadcasted_iota(jnp.int32, (R, B), 1)
    oh = (seg_ids == b_col).astype(jnp.float32)
    g = jax.lax.dot(oh, mi_ref[...], preferred_element_type=jnp.float32)
    mean = g[:, 0:1]
    inv = g[:, 1:2]
    out_ref[...] = (x - mean) * inv * w_ref[...] + bias_ref[...]


@jax.jit
def kernel(x, weight, bias, batch):
    b32 = batch.astype(jnp.int32)
    brow = b32.reshape(N // R, 1, R)
    bcol = b32.reshape(N, 1)
    w2 = weight.reshape(1, D)
    bias2 = bias.reshape(1, D)
    grid = N // R

    acc, mi = pl.pallas_call(
        _pass1_body,
        grid=(grid,),
        in_specs=[
            pl.BlockSpec((R, D), lambda i: (i, 0)),
            pl.BlockSpec((1, 1, R), lambda i: (i, 0, 0)),
        ],
        out_specs=[
            pl.BlockSpec((B, 4), lambda i: (0, 0)),
            pl.BlockSpec((B, 2), lambda i: (0, 0)),
        ],
        out_shape=[
            jax.ShapeDtypeStruct((B, 4), jnp.float32),
            jax.ShapeDtypeStruct((B, 2), jnp.float32),
        ],
    )(x, brow)

    out = pl.pallas_call(
        _pass2_body,
        grid=(grid,),
        in_specs=[
            pl.BlockSpec((R, D), lambda i: (i, 0)),
            pl.BlockSpec((R, 1), lambda i: (i, 0)),
            pl.BlockSpec((1, D), lambda i: (0, 0)),
            pl.BlockSpec((1, D), lambda i: (0, 0)),
            pl.BlockSpec((B, 2), lambda i: (0, 0)),
        ],
        out_specs=pl.BlockSpec((R, D), lambda i: (i, 0)),
        out_shape=jax.ShapeDtypeStruct((N, D), jnp.float32),
    )(x, bcol, w2, bias2, mi)
    return out


# speedup vs baseline: 3.9080x; 1.0271x over previous
import jax
import jax.numpy as jnp
from jax.experimental import pallas as pl

N = 100000
D = 128
B = 512
EPS = 1e-05
R = 5000


def _pass1_body(x_ref, brow_ref, acc_ref, mi_ref):
    i = pl.program_id(0)
    x = x_ref[...]
    rs = jnp.sum(x, axis=1, keepdims=True)
    rq = jnp.sum(x * x, axis=1, keepdims=True)
    ones = jnp.ones((R, 1), jnp.float32)
    vals = jnp.concatenate([ones, rs, rq, jnp.zeros((R, 1), jnp.float32)],
                           axis=1)
    b_row = brow_ref[0]
    seg_ids = jax.lax.broadcasted_iota(jnp.int32, (B, R), 0)
    ohT = (seg_ids == b_row).astype(jnp.float32)

    @pl.when(i == 0)
    def _init():
        acc_ref[...] = jnp.zeros_like(acc_ref)

    acc_ref[...] += jax.lax.dot(ohT, vals, preferred_element_type=jnp.float32)

    @pl.when(i == pl.num_programs(0) - 1)
    def _finalize():
        acc = acc_ref[...]
        cnt = acc[:, 0:1]
        s = acc[:, 1:2]
        q = acc[:, 2:3]
        norm = jnp.maximum(cnt, 1.0) * float(D)
        mean = s / norm
        var = jnp.maximum(q / norm - mean * mean, 0.0)
        inv = 1.0 / (jnp.sqrt(var) + EPS)
        mi_ref[...] = jnp.concatenate([mean, inv], axis=1)


def _pass2_body(x_ref, bcol_ref, w_ref, bias_ref, mi_ref, out_ref):
    x = x_ref[...]
    b_col = bcol_ref[...]
    seg_ids = jax.lax.broadcasted_iota(jnp.int32, (R, B), 1)
    oh = (seg_ids == b_col).astype(jnp.float32)
    g = jax.lax.dot(oh, mi_ref[...], preferred_element_type=jnp.float32)
    mean = g[:, 0:1]
    inv = g[:, 1:2]
    out_ref[...] = (x - mean) * inv * w_ref[...] + bias_ref[...]


@jax.jit
def kernel(x, weight, bias, batch):
    b32 = batch.astype(jnp.int32)
    brow = b32.reshape(N // R, 1, R)
    bcol = b32.reshape(N, 1)
    w2 = weight.reshape(1, D)
    bias2 = bias.reshape(1, D)
    grid = N // R

    acc, mi = pl.pallas_call(
        _pass1_body,
        grid=(grid,),
        in_specs=[
            pl.BlockSpec((R, D), lambda i: (i, 0)),
            pl.BlockSpec((1, 1, R), lambda i: (i, 0, 0)),
        ],
        out_specs=[
            pl.BlockSpec((B, 4), lambda i: (0, 0)),
            pl.BlockSpec((B, 2), lambda i: (0, 0)),
        ],
        out_shape=[
            jax.ShapeDtypeStruct((B, 4), jnp.float32),
            jax.ShapeDtypeStruct((B, 2), jnp.float32),
        ],
    )(x, brow)

    out = pl.pallas_call(
        _pass2_body,
        grid=(grid,),
        in_specs=[
            pl.BlockSpec((R, D), lambda i: (i, 0)),
            pl.BlockSpec((R, 1), lambda i: (i, 0)),
            pl.BlockSpec((1, D), lambda i: (0, 0)),
            pl.BlockSpec((1, D), lambda i: (0, 0)),
            pl.BlockSpec((B, 2), lambda i: (0, 0)),
        ],
        out_specs=pl.BlockSpec((R, D), lambda i: (i, 0)),
        out_shape=jax.ShapeDtypeStruct((N, D), jnp.float32),
    )(x, bcol, w2, bias2, mi)
    return out


# speedup vs baseline: 4.0533x; 1.0372x over previous
import jax
import jax.numpy as jnp
from jax.experimental import pallas as pl

N = 100000
D = 128
B = 512
EPS = 1e-05
R = 10000


def _pass1_body(x_ref, brow_ref, acc_ref, mi_ref):
    i = pl.program_id(0)
    x = x_ref[...]
    rs = jnp.sum(x, axis=1, keepdims=True)
    rq = jnp.sum(x * x, axis=1, keepdims=True)
    ones = jnp.ones((R, 1), jnp.float32)
    vals = jnp.concatenate([ones, rs, rq, jnp.zeros((R, 1), jnp.float32)],
                           axis=1)
    b_row = brow_ref[0]
    seg_ids = jax.lax.broadcasted_iota(jnp.int32, (B, R), 0)
    ohT = (seg_ids == b_row).astype(jnp.float32)

    @pl.when(i == 0)
    def _init():
        acc_ref[...] = jnp.zeros_like(acc_ref)

    acc_ref[...] += jax.lax.dot(ohT, vals, preferred_element_type=jnp.float32)

    @pl.when(i == pl.num_programs(0) - 1)
    def _finalize():
        acc = acc_ref[...]
        cnt = acc[:, 0:1]
        s = acc[:, 1:2]
        q = acc[:, 2:3]
        norm = jnp.maximum(cnt, 1.0) * float(D)
        mean = s / norm
        var = jnp.maximum(q / norm - mean * mean, 0.0)
        inv = 1.0 / (jnp.sqrt(var) + EPS)
        mi_ref[...] = jnp.concatenate([mean, inv], axis=1)


def _pass2_body(x_ref, bcol_ref, w_ref, bias_ref, mi_ref, out_ref):
    x = x_ref[...]
    b_col = bcol_ref[...]
    seg_ids = jax.lax.broadcasted_iota(jnp.int32, (R, B), 1)
    oh = (seg_ids == b_col).astype(jnp.float32)
    g = jax.lax.dot(oh, mi_ref[...], preferred_element_type=jnp.float32)
    mean = g[:, 0:1]
    inv = g[:, 1:2]
    out_ref[...] = (x - mean) * inv * w_ref[...] + bias_ref[...]


@jax.jit
def kernel(x, weight, bias, batch):
    b32 = batch.astype(jnp.int32)
    brow = b32.reshape(N // R, 1, R)
    bcol = b32.reshape(N, 1)
    w2 = weight.reshape(1, D)
    bias2 = bias.reshape(1, D)
    grid = N // R

    acc, mi = pl.pallas_call(
        _pass1_body,
        grid=(grid,),
        in_specs=[
            pl.BlockSpec((R, D), lambda i: (i, 0)),
            pl.BlockSpec((1, 1, R), lambda i: (i, 0, 0)),
        ],
        out_specs=[
            pl.BlockSpec((B, 4), lambda i: (0, 0)),
            pl.BlockSpec((B, 2), lambda i: (0, 0)),
        ],
        out_shape=[
            jax.ShapeDtypeStruct((B, 4), jnp.float32),
            jax.ShapeDtypeStruct((B, 2), jnp.float32),
        ],
    )(x, brow)

    out = pl.pallas_call(
        _pass2_body,
        grid=(grid,),
        in_specs=[
            pl.BlockSpec((R, D), lambda i: (i, 0)),
            pl.BlockSpec((R, 1), lambda i: (i, 0)),
            pl.BlockSpec((1, D), lambda i: (0, 0)),
            pl.BlockSpec((1, D), lambda i: (0, 0)),
            pl.BlockSpec((B, 2), lambda i: (0, 0)),
        ],
        out_specs=pl.BlockSpec((R, D), lambda i: (i, 0)),
        out_shape=jax.ShapeDtypeStruct((N, D), jnp.float32),
    )(x, bcol, w2, bias2, mi)
    return out
